# baseline (speedup 1.0000x reference)
import functools

import jax
import jax.numpy as jnp
from jax import lax
from jax.experimental import pallas as pl
from jax.experimental.pallas import tpu as pltpu
from jax.experimental.pallas import tpu_sc as plsc

_BETA = 0.2
_L = 16
_NC = 2
_NS = 16
_NW = _NC * _NS
_K = 128
_CPP = 64


def _sc_update(P, D, N, m2, idx, v2, out2,
               idx_v, winner_v, wj_v, stage_v, sj_v, rows_m, rows_v1, rows_v2,
               cbuf0, cbuf1, seml0, seml1, sems0, sems1, sem_m, sem_1, sem_2):
    PW = P // _NW
    R = 2 * PW
    wid = lax.axis_index("s") * _NC + lax.axis_index("c")
    pbase = wid * PW
    base = 2 * pbase
    lane = lax.iota(jnp.int32, _L)
    H = D // 2

    RLIN = (PW - 8) // 8 * 8
    NB = PW - RLIN
    hn = (8 - pbase % 8) % 8
    pb_lin = pl.multiple_of(pbase + hn, 8)
    ncp = RLIN // _CPP

    bl = jnp.where(lane < hn, pbase + lane, pb_lin + RLIN + (lane - hn))
    bl = jnp.where(lane < NB, bl, pbase)
    stage_v[3, pl.ds(0, _L)] = bl

    def bpad(q, carry):
        stage_v[3, pl.ds(q * _L, _L)] = jnp.full((_L,), 0, jnp.int32) + pbase
        return carry

    lax.fori_loop(1, _K // _L, bpad, jnp.int32(0))
    pltpu.async_copy(m2.at[stage_v.at[3]], rows_m, sem_m).wait()
    pltpu.sync_copy(rows_m, out2.at[stage_v.at[3]])

    def crange(c):
        return pl.ds(pb_lin + c * _CPP, _CPP)

    npairs2 = ncp // 2

    def cpair(it, carry):
        c0 = it * 2
        l0 = pltpu.async_copy(m2.at[crange(c0)], cbuf0, seml0)
        l1 = pltpu.async_copy(m2.at[crange(c0 + 1)], cbuf1, seml1)
        l0.wait()
        s0 = pltpu.async_copy(cbuf0, out2.at[crange(c0)], sems0)
        l1.wait()
        s1 = pltpu.async_copy(cbuf1, out2.at[crange(c0 + 1)], sems1)
        s0.wait()
        s1.wait()
        return carry

    lax.fori_loop(0, npairs2, cpair, jnp.int32(0))
    if ncp % 2:
        c0 = ncp - 1
        pltpu.async_copy(m2.at[crange(c0)], cbuf0, seml0).wait()
        pltpu.async_copy(cbuf0, out2.at[crange(c0)], sems0).wait()

    WPAD = (R + _L - 1) // _L
    neg1 = jnp.full((_L,), -1, jnp.int32)

    def winit(c, carry):
        winner_v[pl.ds(c * _L, _L)] = neg1
        return carry

    lax.fori_loop(0, WPAD, winit, jnp.int32(0))

    pltpu.sync_copy(idx, idx_v)

    nch = N // _L

    def p1(c, carry):
        jv = idx_v[pl.ds(c * _L, _L)]
        pos = lane + c * _L
        loc = jv - base
        m = (loc >= 0) & (loc < R)
        locc = jnp.where(m, loc, 0)
        _, lastm = plsc.scan_count(jv, mask=m)
        plsc.store_scatter(winner_v, [locc], pos, mask=m & lastm)
        return carry

    lax.fori_loop(0, nch, p1, jnp.int32(0))

    def p2(c, cnt):
        jv = idx_v[pl.ds(c * _L, _L)]
        pos = lane + c * _L
        loc = jv - base
        m = (loc >= 0) & (loc < R)
        locc = jnp.where(m, loc, 0)
        rb = plsc.load_gather(winner_v, [locc], mask=m)
        win = m & (rb == pos)
        sib = plsc.load_gather(winner_v, [locc ^ 1], mask=m)
        h = jv & 1
        lead = win & ((h == 0) | (sib < 0))
        plsc.store_compressed(wj_v.at[pl.ds(cnt, _L)], pos, mask=lead)
        return cnt + jnp.sum(lead.astype(jnp.int32))

    cnt = lax.fori_loop(0, nch, p2, jnp.int32(0))

    lastp = jnp.full((_L,), 0, jnp.int32) + jnp.maximum(cnt - 1, 0)
    lastj = plsc.load_gather(wj_v, [lastp])

    def pad(q, carry):
        wj_v[pl.ds(cnt + q * _L, _L)] = lastj
        return carry

    lax.fori_loop(0, _K // _L, pad, jnp.int32(0))

    nt = (cnt + _K - 1) // _K

    def p3(t, carry):
        off = t * _K

        def st(q, c2):
            wjq = wj_v[pl.ds(off + q * _L, _L)]
            iq = plsc.load_gather(idx_v, [wjq])
            pq = iq >> 1
            eloc = 2 * pq - base
            jev = plsc.load_gather(winner_v, [eloc])
            jov = plsc.load_gather(winner_v, [eloc + 1])
            s = pl.ds(q * _L, _L)
            stage_v[0, s] = pq
            stage_v[1, s] = jnp.maximum(jev, 0) >> 1
            stage_v[2, s] = jnp.maximum(jov, 0) >> 1
            sj_v[0, s] = jev
            sj_v[1, s] = jov
            return c2

        lax.fori_loop(0, _K // _L, st, jnp.int32(0), unroll=True)

        gm = pltpu.async_copy(m2.at[stage_v.at[0]], rows_m, sem_m)
        g1 = pltpu.async_copy(v2.at[stage_v.at[1]], rows_v1, sem_1)
        g2 = pltpu.async_copy(v2.at[stage_v.at[2]], rows_v2, sem_2)
        gm.wait()
        g1.wait()
        g2.wait()

        def bgroup(g, c2):
            s = pl.ds(g * _L, _L)
            jev16 = sj_v[0, s]
            jov16 = sj_v[1, s]
            for r2 in range(_L):
                r = g * _L + r2
                je = jev16[r2]
                jo = jov16[r2]
                me = je >= 0
                mo = jo >= 0
                o1 = (je & 1) * H
                o2 = (jo & 1) * H
                for q in range(H // _L):
                    se = pl.ds(q * _L, _L)
                    a = rows_m[r, se]
                    v = rows_v1[r, pl.ds(o1 + q * _L, _L)]
                    rows_m[r, se] = jnp.where(
                        me, a * _BETA + v * (1.0 - _BETA), a)
                    so = pl.ds(H + q * _L, _L)
                    a2 = rows_m[r, so]
                    w = rows_v2[r, pl.ds(o2 + q * _L, _L)]
                    rows_m[r, so] = jnp.where(
                        mo, a2 * _BETA + w * (1.0 - _BETA), a2)
            return c2

        lax.fori_loop(0, _K // _L, bgroup, jnp.int32(0))
        pltpu.sync_copy(rows_m, out2.at[stage_v.at[0]])
        return carry

    lax.fori_loop(0, nt, p3, jnp.int32(0))


def kernel(mem, idx, val):
    M, Dm = mem.shape
    N = idx.shape[0]
    P, D = M // 2, Dm * 2
    assert P % _NW == 0 and ((P // _NW) - 8) // 8 * 8 % _CPP == 0
    assert N % _L == 0 and Dm % _L == 0

    mesh = plsc.VectorSubcoreMesh(core_axis_name="c", subcore_axis_name="s")
    cap = N + _K + _L
    wpad = ((2 * P // _NW + _L - 1) // _L) * _L
    run = pl.kernel(
        functools.partial(_sc_update, P, D, N),
        out_type=jax.ShapeDtypeStruct((P, D), jnp.float32),
        mesh=mesh,
        compiler_params=pltpu.CompilerParams(use_tc_tiling_on_sc=True,
                                             needs_layout_passes=False),
        scratch_types=[
            pltpu.VMEM((N,), jnp.int32),
            pltpu.VMEM((wpad,), jnp.int32),
            pltpu.VMEM((cap,), jnp.int32),
            pltpu.VMEM((4, _K), jnp.int32),
            pltpu.VMEM((2, _K), jnp.int32),
            pltpu.VMEM((_K, D), jnp.float32),
            pltpu.VMEM((_K, D), jnp.float32),
            pltpu.VMEM((_K, D), jnp.float32),
            pltpu.VMEM((_CPP, D), jnp.float32),
            pltpu.VMEM((_CPP, D), jnp.float32),
            pltpu.SemaphoreType.DMA,
            pltpu.SemaphoreType.DMA,
            pltpu.SemaphoreType.DMA,
            pltpu.SemaphoreType.DMA,
            pltpu.SemaphoreType.DMA,
            pltpu.SemaphoreType.DMA,
            pltpu.SemaphoreType.DMA,
        ],
    )
    m2 = mem.reshape(P, D)
    v2 = val.reshape(N // 2, D)
    out2 = run(m2, idx.astype(jnp.int32), v2)
    return out2.reshape(M, Dm)


# speedup vs baseline: 1.0375x; 1.0375x over previous
import functools

import jax
import jax.numpy as jnp
from jax import lax
from jax.experimental import pallas as pl
from jax.experimental.pallas import tpu as pltpu
from jax.experimental.pallas import tpu_sc as plsc

_BETA = 0.2
_L = 16
_NC = 2
_NS = 16
_NW = _NC * _NS
_K = 128
_CPR = 6250
_NBUF = 4


def _copy_body(P, x_ref, o_ref, bufs, lsems, ssems):
    nc = P // _CPR

    def rng(c):
        return pl.ds(c * _CPR, _CPR)

    def load(c, b):
        return pltpu.make_async_copy(x_ref.at[rng(c)], bufs.at[b], lsems.at[b])

    def store(c, b):
        return pltpu.make_async_copy(bufs.at[b], o_ref.at[rng(c)], ssems.at[b])

    for b in range(_NBUF):
        load(b, b).start()
    for c in range(nc):
        b = c % _NBUF
        load(c, b).wait()
        store(c, b).start()
        nxt = c + _NBUF
        if nxt < nc:
            store(c, b).wait()
            load(nxt, b).start()
    for c in range(max(nc - _NBUF, 0), nc):
        store(c, c % _NBUF).wait()


def _tc_copy(m2):
    P, D = m2.shape
    return pl.pallas_call(
        functools.partial(_copy_body, P),
        in_specs=[pl.BlockSpec(memory_space=pl.ANY)],
        out_specs=pl.BlockSpec(memory_space=pl.ANY),
        scratch_shapes=[
            pltpu.VMEM((_NBUF, _CPR, D), jnp.float32),
            pltpu.SemaphoreType.DMA((_NBUF,)),
            pltpu.SemaphoreType.DMA((_NBUF,)),
        ],
        out_shape=jax.ShapeDtypeStruct((P, D), jnp.float32),
    )(m2)


def _sc_update(P, D, N, idx, v2, y,
               idx_v, winner_v, wj_v, stage_v, sj_v, rows_m, rows_v1, rows_v2,
               sem_m, sem_1, sem_2):
    PW = P // _NW
    R = 2 * PW
    wid = lax.axis_index("s") * _NC + lax.axis_index("c")
    pbase = wid * PW
    base = 2 * pbase
    lane = lax.iota(jnp.int32, _L)
    H = D // 2

    WPAD = (R + _L - 1) // _L
    neg1 = jnp.full((_L,), -1, jnp.int32)

    def winit(c, carry):
        winner_v[pl.ds(c * _L, _L)] = neg1
        return carry

    lax.fori_loop(0, WPAD, winit, jnp.int32(0))

    pltpu.sync_copy(idx, idx_v)

    nch = N // _L

    def p1(c, carry):
        jv = idx_v[pl.ds(c * _L, _L)]
        pos = lane + c * _L
        loc = jv - base
        m = (loc >= 0) & (loc < R)
        locc = jnp.where(m, loc, 0)
        _, lastm = plsc.scan_count(jv, mask=m)
        plsc.store_scatter(winner_v, [locc], pos, mask=m & lastm)
        return carry

    lax.fori_loop(0, nch, p1, jnp.int32(0))

    def p2(c, cnt):
        jv = idx_v[pl.ds(c * _L, _L)]
        pos = lane + c * _L
        loc = jv - base
        m = (loc >= 0) & (loc < R)
        locc = jnp.where(m, loc, 0)
        rb = plsc.load_gather(winner_v, [locc], mask=m)
        win = m & (rb == pos)
        sib = plsc.load_gather(winner_v, [locc ^ 1], mask=m)
        h = jv & 1
        lead = win & ((h == 0) | (sib < 0))
        plsc.store_compressed(wj_v.at[pl.ds(cnt, _L)], pos, mask=lead)
        return cnt + jnp.sum(lead.astype(jnp.int32))

    cnt = lax.fori_loop(0, nch, p2, jnp.int32(0))

    lastp = jnp.full((_L,), 0, jnp.int32) + jnp.maximum(cnt - 1, 0)
    lastj = plsc.load_gather(wj_v, [lastp])

    def pad(q, carry):
        wj_v[pl.ds(cnt + q * _L, _L)] = lastj
        return carry

    lax.fori_loop(0, _K // _L, pad, jnp.int32(0))

    nt = (cnt + _K - 1) // _K

    def p3(t, carry):
        off = t * _K

        def st(q, c2):
            wjq = wj_v[pl.ds(off + q * _L, _L)]
            iq = plsc.load_gather(idx_v, [wjq])
            pq = iq >> 1
            eloc = 2 * pq - base
            jev = plsc.load_gather(winner_v, [eloc])
            jov = plsc.load_gather(winner_v, [eloc + 1])
            s = pl.ds(q * _L, _L)
            stage_v[0, s] = pq
            stage_v[1, s] = jnp.maximum(jev, 0) >> 1
            stage_v[2, s] = jnp.maximum(jov, 0) >> 1
            sj_v[0, s] = jev
            sj_v[1, s] = jov
            return c2

        lax.fori_loop(0, _K // _L, st, jnp.int32(0), unroll=True)

        gm = pltpu.async_copy(y.at[stage_v.at[0]], rows_m, sem_m)
        g1 = pltpu.async_copy(v2.at[stage_v.at[1]], rows_v1, sem_1)
        g2 = pltpu.async_copy(v2.at[stage_v.at[2]], rows_v2, sem_2)
        gm.wait()
        g1.wait()
        g2.wait()

        def bgroup(g, c2):
            s = pl.ds(g * _L, _L)
            jev16 = sj_v[0, s]
            jov16 = sj_v[1, s]
            for r2 in range(_L):
                r = g * _L + r2
                je = jev16[r2]
                jo = jov16[r2]
                me = je >= 0
                mo = jo >= 0
                o1 = (je & 1) * H
                o2 = (jo & 1) * H
                for q in range(H // _L):
                    se = pl.ds(q * _L, _L)
                    a = rows_m[r, se]
                    v = rows_v1[r, pl.ds(o1 + q * _L, _L)]
                    rows_m[r, se] = jnp.where(
                        me, a * _BETA + v * (1.0 - _BETA), a)
                    so = pl.ds(H + q * _L, _L)
                    a2 = rows_m[r, so]
                    w = rows_v2[r, pl.ds(o2 + q * _L, _L)]
                    rows_m[r, so] = jnp.where(
                        mo, a2 * _BETA + w * (1.0 - _BETA), a2)
            return c2

        lax.fori_loop(0, _K // _L, bgroup, jnp.int32(0))
        pltpu.sync_copy(rows_m, y.at[stage_v.at[0]])
        return carry

    lax.fori_loop(0, nt, p3, jnp.int32(0))


def kernel(mem, idx, val):
    M, Dm = mem.shape
    N = idx.shape[0]
    P, D = M // 2, Dm * 2
    assert P % _NW == 0 and P % _CPR == 0
    assert N % _L == 0 and Dm % _L == 0

    mesh = plsc.VectorSubcoreMesh(core_axis_name="c", subcore_axis_name="s")
    cap = N + _K + _L
    wpad = ((2 * P // _NW + _L - 1) // _L) * _L
    run = pl.kernel(
        functools.partial(_sc_update, P, D, N),
        out_type=(),
        mesh=mesh,
        compiler_params=pltpu.CompilerParams(use_tc_tiling_on_sc=False,
                                             needs_layout_passes=False),
        scratch_types=[
            pltpu.VMEM((N,), jnp.int32),
            pltpu.VMEM((wpad,), jnp.int32),
            pltpu.VMEM((cap,), jnp.int32),
            pltpu.VMEM((3, _K), jnp.int32),
            pltpu.VMEM((2, _K), jnp.int32),
            pltpu.VMEM((_K, D), jnp.float32),
            pltpu.VMEM((_K, D), jnp.float32),
            pltpu.VMEM((_K, D), jnp.float32),
            pltpu.SemaphoreType.DMA,
            pltpu.SemaphoreType.DMA,
            pltpu.SemaphoreType.DMA,
        ],
    )
    m2 = mem.reshape(P, D)
    v2 = val.reshape(N // 2, D)
    y = _tc_copy(m2)
    y_ref = jax.new_ref(y)
    run(idx.astype(jnp.int32), v2, y_ref)
    return y_ref[...].reshape(M, Dm)


# speedup vs baseline: 1.5390x; 1.4834x over previous
import functools

import jax
import jax.numpy as jnp
from jax import lax
from jax.experimental import pallas as pl
from jax.experimental.pallas import tpu as pltpu
from jax.experimental.pallas import tpu_sc as plsc

_BETA = 0.2
_L = 16
_NC = 2
_NS = 16
_NW = _NC * _NS
_K = 128
_BR = 5000


def _copy_body(x_ref, o_ref):
    o_ref[...] = x_ref[...]


def _tc_copy(mem):
    M, D = mem.shape
    rows = M * D // 128
    x = mem.reshape(rows, 128)
    y = pl.pallas_call(
        _copy_body,
        grid=(rows // _BR,),
        in_specs=[pl.BlockSpec((_BR, 128), lambda i: (i, 0))],
        out_specs=pl.BlockSpec((_BR, 128), lambda i: (i, 0)),
        out_shape=jax.ShapeDtypeStruct((rows, 128), jnp.float32),
    )(x)
    return y.reshape(M, D)


def _sc_update(M, D, N, idx, val, y,
               idx_v, winner_v, wj_v, stage_v, rows_a, rows_b, sem_a, sem_b):
    R = M // _NW
    wid = lax.axis_index("s") * _NC + lax.axis_index("c")
    base = wid * R
    lane = lax.iota(jnp.int32, _L)

    pltpu.sync_copy(idx, idx_v)

    nch = N // _L

    def p1(c, carry):
        jv = idx_v[pl.ds(c * _L, _L)]
        pos = lane + c * _L
        loc = jv - base
        m = (loc >= 0) & (loc < R)
        locc = jnp.where(m, loc, 0)
        _, lastm = plsc.scan_count(jv, mask=m)
        plsc.store_scatter(winner_v, [locc], pos, mask=m & lastm)
        return carry

    lax.fori_loop(0, nch, p1, jnp.int32(0))

    def p2(c, cnt):
        jv = idx_v[pl.ds(c * _L, _L)]
        pos = lane + c * _L
        loc = jv - base
        m = (loc >= 0) & (loc < R)
        locc = jnp.where(m, loc, 0)
        rb = plsc.load_gather(winner_v, [locc], mask=m)
        win = m & (rb == pos)
        plsc.store_compressed(wj_v.at[pl.ds(cnt, _L)], pos, mask=win)
        return cnt + jnp.sum(win.astype(jnp.int32))

    cnt = lax.fori_loop(0, nch, p2, jnp.int32(0))

    lastp = jnp.full((_L,), 0, jnp.int32) + jnp.maximum(cnt - 1, 0)
    lastj = plsc.load_gather(wj_v, [lastp])

    def pad(q, carry):
        wj_v[pl.ds(cnt + q * _L, _L)] = lastj
        return carry

    lax.fori_loop(0, _K // _L, pad, jnp.int32(0))

    nt = (cnt + _K - 1) // _K

    def p3(t, carry):
        off = t * _K

        def st(q, c2):
            wjq = wj_v[pl.ds(off + q * _L, _L)]
            stage_v[0, pl.ds(q * _L, _L)] = plsc.load_gather(idx_v, [wjq])
            stage_v[1, pl.ds(q * _L, _L)] = wjq
            return c2

        lax.fori_loop(0, _K // _L, st, jnp.int32(0), unroll=True)

        ga = pltpu.async_copy(y.at[stage_v.at[0]], rows_a, sem_a)
        gb = pltpu.async_copy(val.at[stage_v.at[1]], rows_b, sem_b)
        ga.wait()
        gb.wait()

        def blend(r, c2):
            for q in range(D // _L):
                s = pl.ds(q * _L, _L)
                rows_a[r, s] = (rows_a[r, s] * _BETA
                                + rows_b[r, s] * (1.0 - _BETA))
            return c2

        lax.fori_loop(0, _K, blend, jnp.int32(0))
        pltpu.sync_copy(rows_a, y.at[stage_v.at[0]])
        return carry

    lax.fori_loop(0, nt, p3, jnp.int32(0))


def kernel(mem, idx, val):
    M, D = mem.shape
    N = idx.shape[0]
    assert M % _NW == 0 and (M * D // 128) % _BR == 0
    assert N % _L == 0 and D % _L == 0

    mesh = plsc.VectorSubcoreMesh(core_axis_name="c", subcore_axis_name="s")
    cap = N + _K + _L
    run = pl.kernel(
        functools.partial(_sc_update, M, D, N),
        out_type=(),
        mesh=mesh,
        compiler_params=pltpu.CompilerParams(use_tc_tiling_on_sc=False,
                                             needs_layout_passes=False),
        scratch_types=[
            pltpu.VMEM((N,), jnp.int32),
            pltpu.VMEM((M // _NW,), jnp.int32),
            pltpu.VMEM((cap,), jnp.int32),
            pltpu.VMEM((2, _K), jnp.int32),
            pltpu.VMEM((_K, D), jnp.float32),
            pltpu.VMEM((_K, D), jnp.float32),
            pltpu.SemaphoreType.DMA,
            pltpu.SemaphoreType.DMA,
        ],
    )
    y = _tc_copy(mem)
    y_ref = jax.new_ref(y)
    run(idx.astype(jnp.int32), val, y_ref)
    return y_ref[...]
